# Initial kernel scaffold; baseline (speedup 1.0000x reference)
import jax
import jax.numpy as jnp
from jax.experimental import pallas as pl


def kernel(x, edge_index, W1, b1, W2, b2):
    raise NotImplementedError("write your pallas kernel here")


# speedup vs baseline: 3.4275x; 3.4275x over previous
import jax
import jax.numpy as jnp
from jax import lax
from jax.experimental import pallas as pl
from jax.experimental.pallas import tpu as pltpu, tpu_sc as plsc

_N = 10000
_E = 320000
_F = 128
_C = 64
_W = 32
_CH = 128
_NCHUNK = _E // _CH
_NP = 10240
_RPT = 640
_R = 400

_mesh = plsc.VectorSubcoreMesh(
    core_axis_name="c", subcore_axis_name="s", num_cores=2, num_subcores=16)


def _deg_body(src_hbm, dst_hbm, zc_hbm, out_hbm, idx_v, ones_v, stage_v, acc_sh):
    c = lax.axis_index("c")
    s = lax.axis_index("s")
    pltpu.sync_copy(zc_hbm.at[pl.ds(0, _RPT)], stage_v)
    pltpu.sync_copy(stage_v, acc_sh.at[pl.ds(s * _RPT, _RPT)])
    for k in range(_CH // 16):
        ones_v[pl.ds(k * 16, 16)] = jnp.ones((16,), jnp.float32)
    plsc.subcore_barrier()

    def count(idx_hbm):
        def body(j, carry):
            chunk = s + 16 * j

            @pl.when(chunk < _NCHUNK)
            def _():
                pltpu.sync_copy(idx_hbm.at[pl.ds(chunk * _CH, _CH)], idx_v)
                pltpu.sync_copy(ones_v, acc_sh.at[idx_v], add=True)
            return carry
        lax.fori_loop(0, (_NCHUNK + 15) // 16, body, 0)

    @pl.when(c == 0)
    def _():
        count(src_hbm)

    @pl.when(c == 1)
    def _():
        count(dst_hbm)

    plsc.subcore_barrier()
    pltpu.sync_copy(acc_sh.at[pl.ds(s * _RPT, _RPT)], stage_v)
    pltpu.sync_copy(stage_v, out_hbm.at[c, pl.ds(s * _RPT, _RPT)])


_deg_call = pl.kernel(
    _deg_body,
    out_type=jax.ShapeDtypeStruct((2, _NP), jnp.float32),
    mesh=_mesh,
    compiler_params=pltpu.CompilerParams(use_tc_tiling_on_sc=False),
    scratch_types=[
        pltpu.VMEM((_CH,), jnp.int32),
        pltpu.VMEM((_CH,), jnp.float32),
        pltpu.VMEM((_RPT,), jnp.float32),
        pltpu.VMEM_SHARED((_NP,), jnp.float32),
    ],
)


def _prop_body(ta_hbm, tb_hbm, src_hbm, dst_hbm, zr_hbm, out_hbm,
               sidx, didx, rows, stage_v, acc_sh, sem):
    c = lax.axis_index("c")
    s = lax.axis_index("s")
    pltpu.sync_copy(zr_hbm.at[pl.ds(0, _RPT)], stage_v)
    pltpu.sync_copy(stage_v, acc_sh.at[pl.ds(s * _RPT, _RPT)])
    plsc.subcore_barrier()

    def run(tbl_hbm):
        def body(j, carry):
            chunk = s + 16 * j

            @pl.when(chunk < _NCHUNK)
            def _():
                base = chunk * _CH
                pltpu.sync_copy(src_hbm.at[pl.ds(base, _CH)], sidx)
                pltpu.sync_copy(dst_hbm.at[pl.ds(base, _CH)], didx)
                pltpu.async_copy(tbl_hbm.at[sidx], rows, sem).wait()
                pltpu.sync_copy(rows, acc_sh.at[didx], add=True)
            return carry
        lax.fori_loop(0, (_NCHUNK + 15) // 16, body, 0)

    @pl.when(c == 0)
    def _():
        run(ta_hbm)

    @pl.when(c == 1)
    def _():
        run(tb_hbm)

    plsc.subcore_barrier()
    pltpu.sync_copy(acc_sh.at[pl.ds(s * _RPT, _RPT)], stage_v)
    pltpu.sync_copy(stage_v, out_hbm.at[c, pl.ds(s * _RPT, _RPT)])


_prop_call = pl.kernel(
    _prop_body,
    out_type=jax.ShapeDtypeStruct((2, _NP, _W), jnp.float32),
    mesh=_mesh,
    compiler_params=pltpu.CompilerParams(use_tc_tiling_on_sc=False),
    scratch_types=[
        pltpu.VMEM((_CH,), jnp.int32),
        pltpu.VMEM((_CH,), jnp.int32),
        pltpu.VMEM((_CH, _W), jnp.float32),
        pltpu.VMEM((_RPT, _W), jnp.float32),
        pltpu.VMEM_SHARED((_NP, _W), jnp.float32),
        pltpu.SemaphoreType.DMA,
    ],
)


def _scale_split_body(x_ref, degt_ref, norms_ref, *xs_refs):
    ns = lax.rsqrt(jnp.maximum(degt_ref[:, 0:1], 1.0))
    nd = lax.rsqrt(jnp.maximum(degt_ref[:, 1:2], 1.0))
    xs = x_ref[...] * ns
    for k in range(4):
        xs_refs[k][...] = xs[:, k * _W:(k + 1) * _W]
    norms_ref[...] = jnp.concatenate([ns, nd], axis=1)


_scale_split_call = pl.pallas_call(
    _scale_split_body,
    grid=(_N // _R,),
    in_specs=[
        pl.BlockSpec((_R, _F), lambda i: (i, 0)),
        pl.BlockSpec((_R, 2), lambda i: (i, 0)),
    ],
    out_specs=[pl.BlockSpec((_R, 2), lambda i: (i, 0))]
    + [pl.BlockSpec((_R, _W), lambda i: (i, 0)) for _ in range(4)],
    out_shape=[jax.ShapeDtypeStruct((_N, 2), jnp.float32)]
    + [jax.ShapeDtypeStruct((_NP, _W), jnp.float32) for _ in range(4)],
)


def _mlp_body(s1a_ref, s1b_ref, norms_ref, w1_ref, b1_ref, w2_ref,
              t2a_ref, t2b_ref):
    agg = jnp.concatenate(
        [s1a_ref[0], s1a_ref[1], s1b_ref[0], s1b_ref[1]], axis=1)
    h = agg * norms_ref[:, 1:2]
    h = jnp.dot(h, w1_ref[...], preferred_element_type=jnp.float32) + b1_ref[...]
    h = jnp.maximum(h, 0.0)
    t2 = jnp.dot(h, w2_ref[...], preferred_element_type=jnp.float32)
    t2 = t2 * norms_ref[:, 0:1]
    t2a_ref[...] = t2[:, :_W]
    t2b_ref[...] = t2[:, _W:]


_mlp_call = pl.pallas_call(
    _mlp_body,
    grid=(_N // _R,),
    in_specs=[
        pl.BlockSpec((2, _R, _W), lambda i: (0, i, 0)),
        pl.BlockSpec((2, _R, _W), lambda i: (0, i, 0)),
        pl.BlockSpec((_R, 2), lambda i: (i, 0)),
        pl.BlockSpec((_F, _F), lambda i: (0, 0)),
        pl.BlockSpec((1, _F), lambda i: (0, 0)),
        pl.BlockSpec((_F, _C), lambda i: (0, 0)),
    ],
    out_specs=[
        pl.BlockSpec((_R, _W), lambda i: (i, 0)),
        pl.BlockSpec((_R, _W), lambda i: (i, 0)),
    ],
    out_shape=[
        jax.ShapeDtypeStruct((_NP, _W), jnp.float32),
        jax.ShapeDtypeStruct((_NP, _W), jnp.float32),
    ],
)


def _final_body(s2_ref, norms_ref, b2_ref, out_ref):
    agg = jnp.concatenate([s2_ref[0], s2_ref[1]], axis=1)
    out_ref[...] = agg * norms_ref[:, 1:2] + b2_ref[...]


_final_call = pl.pallas_call(
    _final_body,
    grid=(_N // _R,),
    in_specs=[
        pl.BlockSpec((2, _R, _W), lambda i: (0, i, 0)),
        pl.BlockSpec((_R, 2), lambda i: (i, 0)),
        pl.BlockSpec((1, _C), lambda i: (0, 0)),
    ],
    out_specs=pl.BlockSpec((_R, _C), lambda i: (i, 0)),
    out_shape=jax.ShapeDtypeStruct((_N, _C), jnp.float32),
)


def kernel(x, edge_index, W1, b1, W2, b2):
    src = edge_index[0].astype(jnp.int32)
    dst = edge_index[1].astype(jnp.int32)
    zc = jnp.zeros((_RPT,), jnp.float32)
    zr = jnp.zeros((_RPT, _W), jnp.float32)

    degs = _deg_call(src, dst, zc)
    norms, xa, xb, xc, xd = _scale_split_call(x, degs[:, :_N].T)
    s1a = _prop_call(xa, xb, src, dst, zr)
    s1b = _prop_call(xc, xd, src, dst, zr)
    t2a, t2b = _mlp_call(s1a, s1b, norms, W1, b1.reshape(1, -1), W2)
    s2 = _prop_call(t2a, t2b, src, dst, zr)
    return _final_call(s2, norms, b2.reshape(1, -1))


# speedup vs baseline: 5.6667x; 1.6533x over previous
import jax
import jax.numpy as jnp
from jax import lax
from jax.experimental import pallas as pl
from jax.experimental.pallas import tpu as pltpu, tpu_sc as plsc

_N = 10000
_E = 320000
_F = 128
_C = 64
_W = 32
_CH = 128
_EP = 327680
_NCHUNK = _EP // _CH
_CPT = _NCHUNK // 16
_SENT = 10239
_NP = 10240
_RPT = 640
_R = 400

_mesh = plsc.VectorSubcoreMesh(
    core_axis_name="c", subcore_axis_name="s", num_cores=2, num_subcores=16)


def _deg_body(src_hbm, dst_hbm, zc_hbm, out_hbm, idx_v, ones_v, stage_v, acc_sh):
    c = lax.axis_index("c")
    s = lax.axis_index("s")
    pltpu.sync_copy(zc_hbm.at[pl.ds(0, _RPT)], stage_v)
    pltpu.sync_copy(stage_v, acc_sh.at[pl.ds(s * _RPT, _RPT)])
    for k in range(_CH // 16):
        ones_v[pl.ds(k * 16, 16)] = jnp.ones((16,), jnp.float32)

    @pl.when(c == 0)
    def _():
        pltpu.sync_copy(src_hbm.at[pl.ds(s * _CPT, _CPT)], idx_v)

    @pl.when(c == 1)
    def _():
        pltpu.sync_copy(dst_hbm.at[pl.ds(s * _CPT, _CPT)], idx_v)

    plsc.subcore_barrier()

    def body(k, carry):
        pltpu.sync_copy(ones_v, acc_sh.at[idx_v.at[k]], add=True)
        return carry

    lax.fori_loop(0, _CPT, body, 0)
    plsc.subcore_barrier()
    pltpu.sync_copy(acc_sh.at[pl.ds(s * _RPT, _RPT)], stage_v)
    pltpu.sync_copy(stage_v, out_hbm.at[c, pl.ds(s * _RPT, _RPT)])


_deg_call = pl.kernel(
    _deg_body,
    out_type=jax.ShapeDtypeStruct((2, _NP), jnp.float32),
    mesh=_mesh,
    compiler_params=pltpu.CompilerParams(use_tc_tiling_on_sc=False),
    scratch_types=[
        pltpu.VMEM((_CPT, _CH), jnp.int32),
        pltpu.VMEM((_CH,), jnp.float32),
        pltpu.VMEM((_RPT,), jnp.float32),
        pltpu.VMEM_SHARED((_NP,), jnp.float32),
    ],
)


def _prop_body(ta_hbm, tb_hbm, src_hbm, dst_hbm, zr_hbm, out_hbm,
               sidx, didx, rows0, rows1, stage_v, acc_sh, sem0, sem1):
    c = lax.axis_index("c")
    s = lax.axis_index("s")
    pltpu.sync_copy(zr_hbm.at[pl.ds(0, _RPT)], stage_v)
    pltpu.sync_copy(stage_v, acc_sh.at[pl.ds(s * _RPT, _RPT)])
    pltpu.sync_copy(src_hbm.at[pl.ds(s * _CPT, _CPT)], sidx)
    pltpu.sync_copy(dst_hbm.at[pl.ds(s * _CPT, _CPT)], didx)
    plsc.subcore_barrier()

    def run(tbl_hbm):
        pltpu.async_copy(tbl_hbm.at[sidx.at[0]], rows0, sem0)

        def body(j, carry):
            k0 = 2 * j
            pltpu.async_copy(tbl_hbm.at[sidx.at[k0 + 1]], rows1, sem1)
            pltpu.make_async_copy(tbl_hbm.at[sidx.at[k0]], rows0, sem0).wait()
            pltpu.sync_copy(rows0, acc_sh.at[didx.at[k0]], add=True)

            @pl.when(j < _CPT // 2 - 1)
            def _():
                pltpu.async_copy(tbl_hbm.at[sidx.at[k0 + 2]], rows0, sem0)

            pltpu.make_async_copy(tbl_hbm.at[sidx.at[k0 + 1]], rows1, sem1).wait()
            pltpu.sync_copy(rows1, acc_sh.at[didx.at[k0 + 1]], add=True)
            return carry

        lax.fori_loop(0, _CPT // 2, body, 0)

    @pl.when(c == 0)
    def _():
        run(ta_hbm)

    @pl.when(c == 1)
    def _():
        run(tb_hbm)

    plsc.subcore_barrier()
    pltpu.sync_copy(acc_sh.at[pl.ds(s * _RPT, _RPT)], stage_v)
    pltpu.sync_copy(stage_v, out_hbm.at[c, pl.ds(s * _RPT, _RPT)])


_prop_call = pl.kernel(
    _prop_body,
    out_type=jax.ShapeDtypeStruct((2, _NP, _W), jnp.float32),
    mesh=_mesh,
    compiler_params=pltpu.CompilerParams(use_tc_tiling_on_sc=False),
    scratch_types=[
        pltpu.VMEM((_CPT, _CH), jnp.int32),
        pltpu.VMEM((_CPT, _CH), jnp.int32),
        pltpu.VMEM((_CH, _W), jnp.float32),
        pltpu.VMEM((_CH, _W), jnp.float32),
        pltpu.VMEM((_RPT, _W), jnp.float32),
        pltpu.VMEM_SHARED((_NP, _W), jnp.float32),
        pltpu.SemaphoreType.DMA,
        pltpu.SemaphoreType.DMA,
    ],
)


def _scale_split_body(x_ref, degt_ref, norms_ref, *xs_refs):
    ns = lax.rsqrt(jnp.maximum(degt_ref[:, 0:1], 1.0))
    nd = lax.rsqrt(jnp.maximum(degt_ref[:, 1:2], 1.0))
    xs = x_ref[...] * ns
    for k in range(4):
        xs_refs[k][...] = xs[:, k * _W:(k + 1) * _W]
    norms_ref[...] = jnp.concatenate([ns, nd], axis=1)


_scale_split_call = pl.pallas_call(
    _scale_split_body,
    grid=(_N // _R,),
    in_specs=[
        pl.BlockSpec((_R, _F), lambda i: (i, 0)),
        pl.BlockSpec((_R, 2), lambda i: (i, 0)),
    ],
    out_specs=[pl.BlockSpec((_R, 2), lambda i: (i, 0))]
    + [pl.BlockSpec((_R, _W), lambda i: (i, 0)) for _ in range(4)],
    out_shape=[jax.ShapeDtypeStruct((_N, 2), jnp.float32)]
    + [jax.ShapeDtypeStruct((_NP, _W), jnp.float32) for _ in range(4)],
)


def _mlp_body(s1a_ref, s1b_ref, norms_ref, w1_ref, b1_ref, w2_ref,
              t2a_ref, t2b_ref):
    agg = jnp.concatenate(
        [s1a_ref[0], s1a_ref[1], s1b_ref[0], s1b_ref[1]], axis=1)
    h = agg * norms_ref[:, 1:2]
    h = jnp.dot(h, w1_ref[...], preferred_element_type=jnp.float32) + b1_ref[...]
    h = jnp.maximum(h, 0.0)
    t2 = jnp.dot(h, w2_ref[...], preferred_element_type=jnp.float32)
    t2 = t2 * norms_ref[:, 0:1]
    t2a_ref[...] = t2[:, :_W]
    t2b_ref[...] = t2[:, _W:]


_mlp_call = pl.pallas_call(
    _mlp_body,
    grid=(_N // _R,),
    in_specs=[
        pl.BlockSpec((2, _R, _W), lambda i: (0, i, 0)),
        pl.BlockSpec((2, _R, _W), lambda i: (0, i, 0)),
        pl.BlockSpec((_R, 2), lambda i: (i, 0)),
        pl.BlockSpec((_F, _F), lambda i: (0, 0)),
        pl.BlockSpec((1, _F), lambda i: (0, 0)),
        pl.BlockSpec((_F, _C), lambda i: (0, 0)),
    ],
    out_specs=[
        pl.BlockSpec((_R, _W), lambda i: (i, 0)),
        pl.BlockSpec((_R, _W), lambda i: (i, 0)),
    ],
    out_shape=[
        jax.ShapeDtypeStruct((_NP, _W), jnp.float32),
        jax.ShapeDtypeStruct((_NP, _W), jnp.float32),
    ],
)


def _final_body(s2_ref, norms_ref, b2_ref, out_ref):
    agg = jnp.concatenate([s2_ref[0], s2_ref[1]], axis=1)
    out_ref[...] = agg * norms_ref[:, 1:2] + b2_ref[...]


_final_call = pl.pallas_call(
    _final_body,
    grid=(_N // _R,),
    in_specs=[
        pl.BlockSpec((2, _R, _W), lambda i: (0, i, 0)),
        pl.BlockSpec((_R, 2), lambda i: (i, 0)),
        pl.BlockSpec((1, _C), lambda i: (0, 0)),
    ],
    out_specs=pl.BlockSpec((_R, _C), lambda i: (i, 0)),
    out_shape=jax.ShapeDtypeStruct((_N, _C), jnp.float32),
)


def kernel(x, edge_index, W1, b1, W2, b2):
    pad = jnp.full((_EP - _E,), _SENT, jnp.int32)
    src = jnp.concatenate([edge_index[0].astype(jnp.int32), pad]).reshape(_NCHUNK, _CH)
    dst = jnp.concatenate([edge_index[1].astype(jnp.int32), pad]).reshape(_NCHUNK, _CH)
    zc = jnp.zeros((_RPT,), jnp.float32)
    zr = jnp.zeros((_RPT, _W), jnp.float32)

    degs = _deg_call(src, dst, zc)
    norms, xa, xb, xc, xd = _scale_split_call(x, degs[:, :_N].T)
    s1a = _prop_call(xa, xb, src, dst, zr)
    s1b = _prop_call(xc, xd, src, dst, zr)
    t2a, t2b = _mlp_call(s1a, s1b, norms, W1, b1.reshape(1, -1), W2)
    s2 = _prop_call(t2a, t2b, src, dst, zr)
    return _final_call(s2, norms, b2.reshape(1, -1))


# speedup vs baseline: 5.9009x; 1.0413x over previous
import jax
import jax.numpy as jnp
from jax import lax
from jax.experimental import pallas as pl
from jax.experimental.pallas import tpu as pltpu, tpu_sc as plsc

_N = 10000
_E = 320000
_F = 128
_C = 64
_W = 32
_CH = 128
_EP = 327680
_NCHUNK = _EP // _CH
_CPT = _NCHUNK // 16
_SENT = 10239
_NP = 10240
_RPT = 640
_R = 400

_mesh = plsc.VectorSubcoreMesh(
    core_axis_name="c", subcore_axis_name="s", num_cores=2, num_subcores=16)


def _deg_body(src_hbm, dst_hbm, zc_hbm, out_hbm, idx_v, ones_v, stage_v, acc_sh):
    c = lax.axis_index("c")
    s = lax.axis_index("s")
    pltpu.sync_copy(zc_hbm.at[pl.ds(0, _RPT)], stage_v)
    pltpu.sync_copy(stage_v, acc_sh.at[pl.ds(s * _RPT, _RPT)])
    for k in range(_CH // 16):
        ones_v[pl.ds(k * 16, 16)] = jnp.ones((16,), jnp.float32)

    @pl.when(c == 0)
    def _():
        pltpu.sync_copy(src_hbm.at[pl.ds(s * _CPT, _CPT)], idx_v)

    @pl.when(c == 1)
    def _():
        pltpu.sync_copy(dst_hbm.at[pl.ds(s * _CPT, _CPT)], idx_v)

    plsc.subcore_barrier()

    def body(k, carry):
        pltpu.sync_copy(ones_v, acc_sh.at[idx_v.at[k]], add=True)
        return carry

    lax.fori_loop(0, _CPT, body, 0)
    plsc.subcore_barrier()
    pltpu.sync_copy(acc_sh.at[pl.ds(s * _RPT, _RPT)], stage_v)
    pltpu.sync_copy(stage_v, out_hbm.at[c, pl.ds(s * _RPT, _RPT)])


_deg_call = pl.kernel(
    _deg_body,
    out_type=jax.ShapeDtypeStruct((2, _NP), jnp.float32),
    mesh=_mesh,
    compiler_params=pltpu.CompilerParams(use_tc_tiling_on_sc=False),
    scratch_types=[
        pltpu.VMEM((_CPT, _CH), jnp.int32),
        pltpu.VMEM((_CH,), jnp.float32),
        pltpu.VMEM((_RPT,), jnp.float32),
        pltpu.VMEM_SHARED((_NP,), jnp.float32),
    ],
)


_NB = 4


def _prop_body(ta_hbm, tb_hbm, src_hbm, dst_hbm, zr_hbm, out_hbm,
               sidx, didx, rows, stage_v, acc_sh, gsems, ssems):
    c = lax.axis_index("c")
    s = lax.axis_index("s")
    pltpu.sync_copy(zr_hbm.at[pl.ds(0, _RPT)], stage_v)
    pltpu.sync_copy(stage_v, acc_sh.at[pl.ds(s * _RPT, _RPT)])
    pltpu.sync_copy(src_hbm.at[pl.ds(s * _CPT, _CPT)], sidx)
    pltpu.sync_copy(dst_hbm.at[pl.ds(s * _CPT, _CPT)], didx)
    plsc.subcore_barrier()

    def run(tbl_hbm):
        def gather(k, b):
            pltpu.async_copy(tbl_hbm.at[sidx.at[k]], rows.at[b], gsems.at[b])

        def wait_gather(k, b):
            pltpu.make_async_copy(tbl_hbm.at[sidx.at[k]], rows.at[b], gsems.at[b]).wait()

        def scatter(k, b):
            pltpu.async_copy(rows.at[b], acc_sh.at[didx.at[k]], ssems.at[b], add=True)

        def wait_scatter(k, b):
            pltpu.make_async_copy(rows.at[b], acc_sh.at[didx.at[k]], ssems.at[b]).wait()

        for b in range(_NB):
            gather(b, b)

        def body(j, carry):
            k = _NB * j
            for b in range(_NB):
                wait_gather(k + b, b)
                scatter(k + b, b)
            for b in range(_NB):
                kn = k + _NB + b

                @pl.when(kn < _CPT)
                def _(b=b, kn=kn):
                    wait_scatter(kn - _NB, b)
                    gather(kn, b)
            return carry

        lax.fori_loop(0, _CPT // _NB, body, 0)
        for b in range(_NB):
            wait_scatter(_CPT - _NB + b, b)

    @pl.when(c == 0)
    def _():
        run(ta_hbm)

    @pl.when(c == 1)
    def _():
        run(tb_hbm)

    plsc.subcore_barrier()
    pltpu.sync_copy(acc_sh.at[pl.ds(s * _RPT, _RPT)], stage_v)
    pltpu.sync_copy(stage_v, out_hbm.at[c, pl.ds(s * _RPT, _RPT)])


_prop_call = pl.kernel(
    _prop_body,
    out_type=jax.ShapeDtypeStruct((2, _NP, _W), jnp.float32),
    mesh=_mesh,
    compiler_params=pltpu.CompilerParams(use_tc_tiling_on_sc=False),
    scratch_types=[
        pltpu.VMEM((_CPT, _CH), jnp.int32),
        pltpu.VMEM((_CPT, _CH), jnp.int32),
        pltpu.VMEM((_NB, _CH, _W), jnp.float32),
        pltpu.VMEM((_RPT, _W), jnp.float32),
        pltpu.VMEM_SHARED((_NP, _W), jnp.float32),
        pltpu.SemaphoreType.DMA((_NB,)),
        pltpu.SemaphoreType.DMA((_NB,)),
    ],
)


def _scale_split_body(x_ref, degt_ref, norms_ref, *xs_refs):
    ns = lax.rsqrt(jnp.maximum(degt_ref[:, 0:1], 1.0))
    nd = lax.rsqrt(jnp.maximum(degt_ref[:, 1:2], 1.0))
    xs = x_ref[...] * ns
    for k in range(4):
        xs_refs[k][...] = xs[:, k * _W:(k + 1) * _W]
    norms_ref[...] = jnp.concatenate([ns, nd], axis=1)


_scale_split_call = pl.pallas_call(
    _scale_split_body,
    grid=(_N // _R,),
    in_specs=[
        pl.BlockSpec((_R, _F), lambda i: (i, 0)),
        pl.BlockSpec((_R, 2), lambda i: (i, 0)),
    ],
    out_specs=[pl.BlockSpec((_R, 2), lambda i: (i, 0))]
    + [pl.BlockSpec((_R, _W), lambda i: (i, 0)) for _ in range(4)],
    out_shape=[jax.ShapeDtypeStruct((_N, 2), jnp.float32)]
    + [jax.ShapeDtypeStruct((_NP, _W), jnp.float32) for _ in range(4)],
)


def _mlp_body(s1a_ref, s1b_ref, norms_ref, w1_ref, b1_ref, w2_ref,
              t2a_ref, t2b_ref):
    agg = jnp.concatenate(
        [s1a_ref[0], s1a_ref[1], s1b_ref[0], s1b_ref[1]], axis=1)
    h = agg * norms_ref[:, 1:2]
    h = jnp.dot(h, w1_ref[...], preferred_element_type=jnp.float32) + b1_ref[...]
    h = jnp.maximum(h, 0.0)
    t2 = jnp.dot(h, w2_ref[...], preferred_element_type=jnp.float32)
    t2 = t2 * norms_ref[:, 0:1]
    t2a_ref[...] = t2[:, :_W]
    t2b_ref[...] = t2[:, _W:]


_mlp_call = pl.pallas_call(
    _mlp_body,
    grid=(_N // _R,),
    in_specs=[
        pl.BlockSpec((2, _R, _W), lambda i: (0, i, 0)),
        pl.BlockSpec((2, _R, _W), lambda i: (0, i, 0)),
        pl.BlockSpec((_R, 2), lambda i: (i, 0)),
        pl.BlockSpec((_F, _F), lambda i: (0, 0)),
        pl.BlockSpec((1, _F), lambda i: (0, 0)),
        pl.BlockSpec((_F, _C), lambda i: (0, 0)),
    ],
    out_specs=[
        pl.BlockSpec((_R, _W), lambda i: (i, 0)),
        pl.BlockSpec((_R, _W), lambda i: (i, 0)),
    ],
    out_shape=[
        jax.ShapeDtypeStruct((_NP, _W), jnp.float32),
        jax.ShapeDtypeStruct((_NP, _W), jnp.float32),
    ],
)


def _final_body(s2_ref, norms_ref, b2_ref, out_ref):
    agg = jnp.concatenate([s2_ref[0], s2_ref[1]], axis=1)
    out_ref[...] = agg * norms_ref[:, 1:2] + b2_ref[...]


_final_call = pl.pallas_call(
    _final_body,
    grid=(_N // _R,),
    in_specs=[
        pl.BlockSpec((2, _R, _W), lambda i: (0, i, 0)),
        pl.BlockSpec((_R, 2), lambda i: (i, 0)),
        pl.BlockSpec((1, _C), lambda i: (0, 0)),
    ],
    out_specs=pl.BlockSpec((_R, _C), lambda i: (i, 0)),
    out_shape=jax.ShapeDtypeStruct((_N, _C), jnp.float32),
)


def kernel(x, edge_index, W1, b1, W2, b2):
    pad = jnp.full((_EP - _E,), _SENT, jnp.int32)
    src = jnp.concatenate([edge_index[0].astype(jnp.int32), pad]).reshape(_NCHUNK, _CH)
    dst = jnp.concatenate([edge_index[1].astype(jnp.int32), pad]).reshape(_NCHUNK, _CH)
    zc = jnp.zeros((_RPT,), jnp.float32)
    zr = jnp.zeros((_RPT, _W), jnp.float32)

    degs = _deg_call(src, dst, zc)
    norms, xa, xb, xc, xd = _scale_split_call(x, degs[:, :_N].T)
    s1a = _prop_call(xa, xb, src, dst, zr)
    s1b = _prop_call(xc, xd, src, dst, zr)
    t2a, t2b = _mlp_call(s1a, s1b, norms, W1, b1.reshape(1, -1), W2)
    s2 = _prop_call(t2a, t2b, src, dst, zr)
    return _final_call(s2, norms, b2.reshape(1, -1))
